# baseline (speedup 1.0000x reference)
import jax
import jax.numpy as jnp
from jax.experimental import pallas as pl
from jax.experimental.pallas import tpu as pltpu

_ROWS_PER_CHUNK = 512


def kernel(x, pe_weight):
    batch = x.shape[0]
    maxlen, d = pe_weight.shape
    rb = _ROWS_PER_CHUNK
    nchunk = maxlen // rb

    def _body(w_hbm, out_hbm, buf, in_sems, out_sems):
        for i in range(nchunk):
            pltpu.make_async_copy(
                w_hbm.at[pl.ds(i * rb, rb)],
                buf.at[pl.ds(i * rb, rb)],
                in_sems.at[i],
            ).start()
        for i in range(nchunk):
            pltpu.make_async_copy(
                w_hbm.at[pl.ds(i * rb, rb)],
                buf.at[pl.ds(i * rb, rb)],
                in_sems.at[i],
            ).wait()
            for b in range(batch):
                pltpu.make_async_copy(
                    buf.at[pl.ds(i * rb, rb)],
                    out_hbm.at[b, pl.ds(i * rb, rb)],
                    out_sems.at[i, b],
                ).start()
        for i in range(nchunk):
            for b in range(batch):
                pltpu.make_async_copy(
                    buf.at[pl.ds(i * rb, rb)],
                    out_hbm.at[b, pl.ds(i * rb, rb)],
                    out_sems.at[i, b],
                ).wait()

    return pl.pallas_call(
        _body,
        in_specs=[pl.BlockSpec(memory_space=pltpu.MemorySpace.HBM)],
        out_specs=pl.BlockSpec(memory_space=pltpu.MemorySpace.HBM),
        out_shape=jax.ShapeDtypeStruct((batch, maxlen, d), pe_weight.dtype),
        scratch_shapes=[
            pltpu.VMEM((maxlen, d), pe_weight.dtype),
            pltpu.SemaphoreType.DMA((nchunk,)),
            pltpu.SemaphoreType.DMA((nchunk, batch)),
        ],
    )(pe_weight)


# speedup vs baseline: 1.0139x; 1.0139x over previous
import jax
import jax.numpy as jnp
from jax.experimental import pallas as pl
from jax.experimental.pallas import tpu as pltpu

_ROWS_PER_CHUNK = 2048


def kernel(x, pe_weight):
    batch = x.shape[0]
    maxlen, d = pe_weight.shape
    rb = _ROWS_PER_CHUNK
    nchunk = maxlen // rb

    def _body(w_hbm, out_hbm, buf, in_sems, out_sems):
        for i in range(nchunk):
            pltpu.make_async_copy(
                w_hbm.at[pl.ds(i * rb, rb)],
                buf.at[pl.ds(i * rb, rb)],
                in_sems.at[i],
            ).start()
        for i in range(nchunk):
            pltpu.make_async_copy(
                w_hbm.at[pl.ds(i * rb, rb)],
                buf.at[pl.ds(i * rb, rb)],
                in_sems.at[i],
            ).wait()
            for b in range(batch):
                pltpu.make_async_copy(
                    buf.at[pl.ds(i * rb, rb)],
                    out_hbm.at[b, pl.ds(i * rb, rb)],
                    out_sems.at[i, b],
                ).start()
        for i in range(nchunk):
            for b in range(batch):
                pltpu.make_async_copy(
                    buf.at[pl.ds(i * rb, rb)],
                    out_hbm.at[b, pl.ds(i * rb, rb)],
                    out_sems.at[i, b],
                ).wait()

    return pl.pallas_call(
        _body,
        in_specs=[pl.BlockSpec(memory_space=pltpu.MemorySpace.HBM)],
        out_specs=pl.BlockSpec(memory_space=pltpu.MemorySpace.HBM),
        out_shape=jax.ShapeDtypeStruct((batch, maxlen, d), pe_weight.dtype),
        scratch_shapes=[
            pltpu.VMEM((maxlen, d), pe_weight.dtype),
            pltpu.SemaphoreType.DMA((nchunk,)),
            pltpu.SemaphoreType.DMA((nchunk, batch)),
        ],
    )(pe_weight)


# speedup vs baseline: 1.0201x; 1.0061x over previous
import jax
import jax.numpy as jnp
from jax.experimental import pallas as pl
from jax.experimental.pallas import tpu as pltpu

_ROWS_PER_CHUNK = 4096


def kernel(x, pe_weight):
    batch = x.shape[0]
    maxlen, d = pe_weight.shape
    rb = _ROWS_PER_CHUNK
    nchunk = maxlen // rb

    def _body(w_hbm, out_hbm, buf, in_sems, out_sems):
        for i in range(nchunk):
            pltpu.make_async_copy(
                w_hbm.at[pl.ds(i * rb, rb)],
                buf.at[pl.ds(i * rb, rb)],
                in_sems.at[i],
            ).start()
        for i in range(nchunk):
            pltpu.make_async_copy(
                w_hbm.at[pl.ds(i * rb, rb)],
                buf.at[pl.ds(i * rb, rb)],
                in_sems.at[i],
            ).wait()
            for b in range(batch):
                pltpu.make_async_copy(
                    buf.at[pl.ds(i * rb, rb)],
                    out_hbm.at[b, pl.ds(i * rb, rb)],
                    out_sems.at[i, b],
                ).start()
        for i in range(nchunk):
            for b in range(batch):
                pltpu.make_async_copy(
                    buf.at[pl.ds(i * rb, rb)],
                    out_hbm.at[b, pl.ds(i * rb, rb)],
                    out_sems.at[i, b],
                ).wait()

    return pl.pallas_call(
        _body,
        in_specs=[pl.BlockSpec(memory_space=pltpu.MemorySpace.HBM)],
        out_specs=pl.BlockSpec(memory_space=pltpu.MemorySpace.HBM),
        out_shape=jax.ShapeDtypeStruct((batch, maxlen, d), pe_weight.dtype),
        scratch_shapes=[
            pltpu.VMEM((maxlen, d), pe_weight.dtype),
            pltpu.SemaphoreType.DMA((nchunk,)),
            pltpu.SemaphoreType.DMA((nchunk, batch)),
        ],
    )(pe_weight)
